# Initial kernel scaffold; baseline (speedup 1.0000x reference)
import jax
import jax.numpy as jnp
from jax.experimental import pallas as pl


def kernel(feat, edges, label_idx, w1, b1, w2, b2):
    raise NotImplementedError("write your pallas kernel here")


# speedup vs baseline: 6.8420x; 6.8420x over previous
import functools

import jax
import jax.numpy as jnp
from jax import lax
from jax.experimental import pallas as pl
from jax.experimental.pallas import tpu as pltpu
from jax.experimental.pallas import tpu_sc as plsc

_B = 128
_LANES = 16


def _matmul1_body(x_ref, w_ref, o_ref):
    h = jnp.dot(x_ref[...], w_ref[...], preferred_element_type=jnp.float32)
    o_ref[0, :, :] = h[:, :32]
    o_ref[1, :, :] = h[:, 32:]


def _mid_body(g_ref, b_ref, w_ref, o_ref):
    h = jnp.concatenate([g_ref[0, :, :], g_ref[1, :, :]], axis=1) + b_ref[...]
    h = jnp.where(h >= 0, h, 0.2 * h)
    nrm = jnp.sqrt(jnp.sum(h * h, axis=1, keepdims=True))
    y = h / jnp.maximum(nrm, 1e-12)
    h2 = jnp.dot(y, w_ref[...], preferred_element_type=jnp.float32)
    o_ref[0, :, :] = h2[:, :32]
    o_ref[1, :, :] = h2[:, 32:]


def _final_body(g_ref, b_ref, o_ref):
    h = jnp.concatenate([g_ref[0, :, :], g_ref[1, :, :]], axis=1) + b_ref[...]
    h = jnp.where(h >= 0, h, 0.2 * h)
    nrm = jnp.sqrt(jnp.sum(h * h, axis=1, keepdims=True))
    o_ref[...] = h / jnp.maximum(nrm, 1e-12)


def _make_sc_agg(n, npad, ept, nblk):
    zr = npad // _LANES
    wr_hi = -(-(n // _LANES) // 8) * 8
    wr_lo = n - (_LANES - 1) * wr_hi
    assert wr_lo > 0 and wr_lo % 8 == 0
    mesh = plsc.VectorSubcoreMesh(core_axis_name="c", subcore_axis_name="s")

    @functools.partial(
        pl.kernel,
        out_type=jax.ShapeDtypeStruct((2 * n, 32), jnp.float32),
        mesh=mesh,
        scratch_types=[
            pltpu.VMEM((_B,), jnp.int32),
            pltpu.VMEM((_B,), jnp.int32),
            pltpu.VMEM((_B, 32), jnp.float32),
            pltpu.VMEM_SHARED((npad, 32), jnp.float32),
            pltpu.SemaphoreType.DMA,
        ],
        compiler_params=pltpu.CompilerParams(use_tc_tiling_on_sc=False),
    )
    def agg(h_hbm, src_hbm, dst_hbm, zeros_hbm, out_hbm, srcv, dstv, rows, acc, sem):
        c = lax.axis_index("c")
        s = lax.axis_index("s")
        pltpu.sync_copy(zeros_hbm, acc.at[pl.ds(s * zr, zr)])
        plsc.subcore_barrier()
        coff = c * n
        ebase = s * ept

        def block(i, carry):
            off = ebase + i * _B
            pltpu.sync_copy(src_hbm.at[pl.ds(off, _B)], srcv)
            pltpu.sync_copy(dst_hbm.at[pl.ds(off, _B)], dstv)
            for j in range(_B // _LANES):
                sl = pl.ds(j * _LANES, _LANES)
                srcv[sl] = srcv[sl] + coff
            pltpu.async_copy(h_hbm.at[srcv], rows, sem).wait()
            pltpu.sync_copy(rows, acc.at[dstv], add=True)
            return carry

        lax.fori_loop(0, nblk, block, 0)
        plsc.subcore_barrier()

        @pl.when(s < _LANES - 1)
        def _():
            pltpu.sync_copy(acc.at[pl.ds(s * wr_hi, wr_hi)],
                            out_hbm.at[pl.ds(c * n + s * wr_hi, wr_hi)])

        @pl.when(s == _LANES - 1)
        def _():
            base = (_LANES - 1) * wr_hi
            pltpu.sync_copy(acc.at[pl.ds(base, wr_lo)],
                            out_hbm.at[pl.ds(c * n + base, wr_lo)])

    return agg


def _make_sc_label_gather(n, lpad):
    lt = lpad // _LANES
    mesh = plsc.VectorSubcoreMesh(core_axis_name="c", subcore_axis_name="s")

    @functools.partial(
        pl.kernel,
        out_type=jax.ShapeDtypeStruct((2 * lpad, 32), jnp.float32),
        mesh=mesh,
        scratch_types=[
            pltpu.VMEM((lt,), jnp.int32),
            pltpu.VMEM((lt, 32), jnp.float32),
            pltpu.SemaphoreType.DMA,
        ],
        compiler_params=pltpu.CompilerParams(use_tc_tiling_on_sc=False),
    )
    def gather(agg_hbm, lab_hbm, out_hbm, labv, lrows, sem):
        c = lax.axis_index("c")
        s = lax.axis_index("s")
        off = s * lt
        pltpu.sync_copy(lab_hbm.at[pl.ds(off, lt)], labv)
        coff = c * n
        for j in range(lt // _LANES):
            sl = pl.ds(j * _LANES, _LANES)
            labv[sl] = labv[sl] + coff
        pltpu.async_copy(agg_hbm.at[labv], lrows, sem).wait()
        pltpu.sync_copy(lrows, out_hbm.at[pl.ds(c * lpad + off, lt)])

    return gather


def kernel(feat, edges, label_idx, w1, b1, w2, b2):
    n, d_in = feat.shape
    e = edges.shape[0]
    l = label_idx.shape[0]
    d_h = w1.shape[1]
    assert d_h == 64 and n % _LANES == 0

    ept = -(-e // (_LANES * _B)) * _B
    epad = _LANES * ept
    nblk = ept // _B
    npad = -(-(n + 1) // _LANES) * _LANES
    src = jnp.concatenate([edges[:, 0], jnp.zeros((epad - e,), jnp.int32)])
    dst = jnp.concatenate([edges[:, 1], jnp.full((epad - e,), n, jnp.int32)])
    zeros_blk = jnp.zeros((npad // _LANES, 32), jnp.float32)

    lpad = -(-l // (_LANES * _LANES)) * _LANES * _LANES
    lab = jnp.concatenate([label_idx, jnp.zeros((lpad - l,), jnp.int32)])

    bm = 400
    h1 = pl.pallas_call(
        _matmul1_body,
        grid=(n // bm,),
        in_specs=[
            pl.BlockSpec((bm, d_in), lambda i: (i, 0)),
            pl.BlockSpec((d_in, d_h), lambda i: (0, 0)),
        ],
        out_specs=pl.BlockSpec((2, bm, 32), lambda i: (0, i, 0)),
        out_shape=jax.ShapeDtypeStruct((2, n, 32), jnp.float32),
    )(feat, w1).reshape(2 * n, 32)

    agg_fn = _make_sc_agg(n, npad, ept, nblk)
    agg1 = agg_fn(h1, src, dst, zeros_blk)

    h2 = pl.pallas_call(
        _mid_body,
        grid=(n // bm,),
        in_specs=[
            pl.BlockSpec((2, bm, 32), lambda i: (0, i, 0)),
            pl.BlockSpec((1, d_h), lambda i: (0, 0)),
            pl.BlockSpec((d_h, d_h), lambda i: (0, 0)),
        ],
        out_specs=pl.BlockSpec((2, bm, 32), lambda i: (0, i, 0)),
        out_shape=jax.ShapeDtypeStruct((2, n, 32), jnp.float32),
    )(agg1.reshape(2, n, 32), b1.reshape(1, d_h), w2).reshape(2 * n, 32)

    agg2 = agg_fn(h2, src, dst, zeros_blk)

    gat = _make_sc_label_gather(n, lpad)(agg2, lab)

    out = pl.pallas_call(
        _final_body,
        grid=(1,),
        in_specs=[
            pl.BlockSpec((2, lpad, 32), lambda i: (0, 0, 0)),
            pl.BlockSpec((1, d_h), lambda i: (0, 0)),
        ],
        out_specs=pl.BlockSpec((lpad, d_h), lambda i: (0, 0)),
        out_shape=jax.ShapeDtypeStruct((lpad, d_h), jnp.float32),
    )(gat.reshape(2, lpad, 32), b2.reshape(1, d_h))

    return out[:l]


# speedup vs baseline: 8.9399x; 1.3066x over previous
import functools

import jax
import jax.numpy as jnp
from jax import lax
from jax.experimental import pallas as pl
from jax.experimental.pallas import tpu as pltpu
from jax.experimental.pallas import tpu_sc as plsc

_B = 128
_LANES = 16


def _matmul1_body(x_ref, w_ref, o_ref):
    h = jnp.dot(x_ref[...], w_ref[...], preferred_element_type=jnp.float32)
    o_ref[0, :, :] = h[:, :32]
    o_ref[1, :, :] = h[:, 32:]


def _mid_body(g_ref, b_ref, w_ref, o_ref):
    h = jnp.concatenate([g_ref[0, :, :], g_ref[1, :, :]], axis=1) + b_ref[...]
    h = jnp.where(h >= 0, h, 0.2 * h)
    nrm = jnp.sqrt(jnp.sum(h * h, axis=1, keepdims=True))
    y = h / jnp.maximum(nrm, 1e-12)
    h2 = jnp.dot(y, w_ref[...], preferred_element_type=jnp.float32)
    o_ref[0, :, :] = h2[:, :32]
    o_ref[1, :, :] = h2[:, 32:]


def _final_body(g_ref, b_ref, o_ref):
    h = jnp.concatenate([g_ref[0, :, :], g_ref[1, :, :]], axis=1) + b_ref[...]
    h = jnp.where(h >= 0, h, 0.2 * h)
    nrm = jnp.sqrt(jnp.sum(h * h, axis=1, keepdims=True))
    o_ref[...] = h / jnp.maximum(nrm, 1e-12)


_SBK = 8
_SBE = _SBK * _B
_PF = 2 * _SBE


def _make_sc_agg(n, npad, ept, nsb):
    zr = npad // _LANES
    wr_hi = -(-(n // _LANES) // 8) * 8
    wr_lo = n - (_LANES - 1) * wr_hi
    assert wr_lo > 0 and wr_lo % 8 == 0
    assert nsb % 2 == 0
    half = _LANES * ept + _PF
    mesh = plsc.VectorSubcoreMesh(core_axis_name="c", subcore_axis_name="s")

    @functools.partial(
        pl.kernel,
        out_type=jax.ShapeDtypeStruct((2 * n, 32), jnp.float32),
        mesh=mesh,
        scratch_types=[
            pltpu.VMEM((_SBE,), jnp.int32),
            pltpu.VMEM((_SBE,), jnp.int32),
            pltpu.VMEM((_SBE,), jnp.int32),
            pltpu.VMEM((_SBE,), jnp.int32),
            pltpu.VMEM((_B,), jnp.int32),
            pltpu.VMEM((_B,), jnp.int32),
            pltpu.VMEM((_B, 32), jnp.float32),
            pltpu.VMEM((_B, 32), jnp.float32),
            pltpu.VMEM_SHARED((npad, 32), jnp.float32),
            pltpu.SemaphoreType.DMA,
            pltpu.SemaphoreType.DMA,
            pltpu.SemaphoreType.DMA,
            pltpu.SemaphoreType.DMA,
            pltpu.SemaphoreType.DMA,
            pltpu.SemaphoreType.DMA,
            pltpu.SemaphoreType.DMA,
            pltpu.SemaphoreType.DMA,
        ],
        compiler_params=pltpu.CompilerParams(use_tc_tiling_on_sc=False),
    )
    def agg(h_hbm, srcoff_hbm, dst_hbm, zeros_hbm, out_hbm,
            sa0, sa1, da0, da1, ix0, ix1, r0, r1, acc,
            qs0, qs1, qd0, qd1, g0, g1, t0, t1):
        c = lax.axis_index("c")
        s = lax.axis_index("s")
        SA, DA, IX, R = [sa0, sa1], [da0, da1], [ix0, ix1], [r0, r1]
        QS, QD, G, T = [qs0, qs1], [qd0, qd1], [g0, g1], [t0, t1]
        sbase = c * half + s * ept
        dbase = s * ept

        def stage_issue(sb, hb):
            pltpu.async_copy(srcoff_hbm.at[pl.ds(sbase + sb * _SBE, _SBE)],
                             SA[hb], QS[hb])
            pltpu.async_copy(dst_hbm.at[pl.ds(dbase + sb * _SBE, _SBE)],
                             DA[hb], QD[hb])

        def stage_wait(hb):
            pltpu.make_async_copy(srcoff_hbm.at[pl.ds(sbase, _SBE)],
                                  SA[hb], QS[hb]).wait()
            pltpu.make_async_copy(dst_hbm.at[pl.ds(dbase, _SBE)],
                                  DA[hb], QD[hb]).wait()

        def gather_issue(hb, k, p):
            pltpu.async_copy(h_hbm.at[SA[hb].at[pl.ds(k * _B, _B)]], R[p], G[p])

        def gather_wait(p):
            pltpu.make_async_copy(h_hbm.at[SA[0].at[pl.ds(0, _B)]],
                                  R[p], G[p]).wait()

        def scatter_issue(p):
            pltpu.async_copy(R[p], acc.at[IX[p]], T[p], add=True)

        def scatter_wait(p):
            pltpu.make_async_copy(R[p], acc.at[IX[p]], T[p]).wait()

        def prep(hb, k, q):
            for j in range(_B // _LANES):
                IX[q][pl.ds(j * _LANES, _LANES)] = (
                    DA[hb][pl.ds(k * _B + j * _LANES, _LANES)])

        pltpu.sync_copy(zeros_hbm, acc.at[pl.ds(s * zr, zr)])
        plsc.subcore_barrier()

        stage_issue(0, 0)
        stage_issue(1, 1)
        trash = jnp.full((_LANES,), n, jnp.int32)
        for j in range(_B // _LANES):
            IX[1][pl.ds(j * _LANES, _LANES)] = trash
        scatter_issue(1)
        stage_wait(0)
        prep(0, 0, 0)
        gather_issue(0, 0, 0)

        def slot(sb, k, hb):
            p = k % 2
            q = 1 - p
            nk = (k + 1) % _SBK
            nh = hb if k < _SBK - 1 else 1 - hb
            gather_wait(p)
            scatter_issue(p)
            if k == _SBK - 1:
                stage_wait(nh)
            scatter_wait(q)
            prep(nh, nk, q)
            if k == _SBK - 1:
                stage_issue(sb + 2, hb)
            gather_issue(nh, nk, q)

        def pair_body(pair, carry):
            for hb in (0, 1):
                for k in range(_SBK):
                    slot(2 * pair + hb, k, hb)
            return carry

        lax.fori_loop(0, nsb // 2, pair_body, 0)

        gather_wait(0)
        scatter_wait(1)
        stage_wait(1)

        plsc.subcore_barrier()

        @pl.when(s < _LANES - 1)
        def _():
            pltpu.sync_copy(acc.at[pl.ds(s * wr_hi, wr_hi)],
                            out_hbm.at[pl.ds(c * n + s * wr_hi, wr_hi)])

        @pl.when(s == _LANES - 1)
        def _():
            base = (_LANES - 1) * wr_hi
            pltpu.sync_copy(acc.at[pl.ds(base, wr_lo)],
                            out_hbm.at[pl.ds(c * n + base, wr_lo)])

    return agg


def _make_sc_label_gather(n, lpad):
    lt = lpad // _LANES
    mesh = plsc.VectorSubcoreMesh(core_axis_name="c", subcore_axis_name="s")

    @functools.partial(
        pl.kernel,
        out_type=jax.ShapeDtypeStruct((2 * lpad, 32), jnp.float32),
        mesh=mesh,
        scratch_types=[
            pltpu.VMEM((lt,), jnp.int32),
            pltpu.VMEM((lt, 32), jnp.float32),
            pltpu.SemaphoreType.DMA,
        ],
        compiler_params=pltpu.CompilerParams(use_tc_tiling_on_sc=False),
    )
    def gather(agg_hbm, lab_hbm, out_hbm, labv, lrows, sem):
        c = lax.axis_index("c")
        s = lax.axis_index("s")
        off = s * lt
        pltpu.sync_copy(lab_hbm.at[pl.ds(off, lt)], labv)
        coff = c * n
        for j in range(lt // _LANES):
            sl = pl.ds(j * _LANES, _LANES)
            labv[sl] = labv[sl] + coff
        pltpu.async_copy(agg_hbm.at[labv], lrows, sem).wait()
        pltpu.sync_copy(lrows, out_hbm.at[pl.ds(c * lpad + off, lt)])

    return gather


def kernel(feat, edges, label_idx, w1, b1, w2, b2):
    n, d_in = feat.shape
    e = edges.shape[0]
    l = label_idx.shape[0]
    d_h = w1.shape[1]
    assert d_h == 64 and n % _LANES == 0

    nsb = 2 * -(-e // (_LANES * _SBE * 2))
    ept = nsb * _SBE
    epad = _LANES * ept
    npad = -(-(n + 1) // _LANES) * _LANES
    src = jnp.concatenate([edges[:, 0], jnp.zeros((epad - e,), jnp.int32)])
    dst = jnp.concatenate([edges[:, 1], jnp.full((epad - e,), n, jnp.int32),
                           jnp.full((_PF,), n, jnp.int32)])
    srcoff = jnp.concatenate([src, jnp.zeros((_PF,), jnp.int32),
                              src + n, jnp.zeros((_PF,), jnp.int32)])
    zeros_blk = jnp.zeros((npad // _LANES, 32), jnp.float32)

    lpad = -(-l // (_LANES * _LANES)) * _LANES * _LANES
    lab = jnp.concatenate([label_idx, jnp.zeros((lpad - l,), jnp.int32)])

    bm = 400
    h1 = pl.pallas_call(
        _matmul1_body,
        grid=(n // bm,),
        in_specs=[
            pl.BlockSpec((bm, d_in), lambda i: (i, 0)),
            pl.BlockSpec((d_in, d_h), lambda i: (0, 0)),
        ],
        out_specs=pl.BlockSpec((2, bm, 32), lambda i: (0, i, 0)),
        out_shape=jax.ShapeDtypeStruct((2, n, 32), jnp.float32),
    )(feat, w1).reshape(2 * n, 32)

    agg_fn = _make_sc_agg(n, npad, ept, nsb)
    agg1 = agg_fn(h1, srcoff, dst, zeros_blk)

    h2 = pl.pallas_call(
        _mid_body,
        grid=(n // bm,),
        in_specs=[
            pl.BlockSpec((2, bm, 32), lambda i: (0, i, 0)),
            pl.BlockSpec((1, d_h), lambda i: (0, 0)),
            pl.BlockSpec((d_h, d_h), lambda i: (0, 0)),
        ],
        out_specs=pl.BlockSpec((2, bm, 32), lambda i: (0, i, 0)),
        out_shape=jax.ShapeDtypeStruct((2, n, 32), jnp.float32),
    )(agg1.reshape(2, n, 32), b1.reshape(1, d_h), w2).reshape(2 * n, 32)

    agg2 = agg_fn(h2, srcoff, dst, zeros_blk)

    gat = _make_sc_label_gather(n, lpad)(agg2, lab)

    out = pl.pallas_call(
        _final_body,
        grid=(1,),
        in_specs=[
            pl.BlockSpec((2, lpad, 32), lambda i: (0, 0, 0)),
            pl.BlockSpec((1, d_h), lambda i: (0, 0)),
        ],
        out_specs=pl.BlockSpec((lpad, d_h), lambda i: (0, 0)),
        out_shape=jax.ShapeDtypeStruct((lpad, d_h), jnp.float32),
    )(gat.reshape(2, lpad, 32), b2.reshape(1, d_h))

    return out[:l]
